# Initial kernel scaffold; baseline (speedup 1.0000x reference)
import jax
import jax.numpy as jnp
from jax.experimental import pallas as pl


def kernel(input, lengths, emb_table):
    raise NotImplementedError("write your pallas kernel here")


# speedup vs baseline: 1.3053x; 1.3053x over previous
import functools

import jax
import jax.numpy as jnp
from jax import lax
from jax.experimental import pallas as pl
from jax.experimental.pallas import tpu as pltpu
from jax.experimental.pallas import tpu_sc as plsc

_C = 128


def _gather_kernel(n_rows, d, n_workers, n_chunks, num_cores):
    @functools.partial(
        pl.kernel,
        mesh=plsc.VectorSubcoreMesh(core_axis_name="c", subcore_axis_name="s"),
        out_type=jax.ShapeDtypeStruct((n_rows, d), jnp.float32),
        compiler_params=pltpu.CompilerParams(use_tc_tiling_on_sc=False),
        scratch_types=[
            pltpu.VMEM((n_chunks, _C), jnp.int32),
            pltpu.VMEM((_C, d), jnp.float32),
            pltpu.SemaphoreType.DMA,
        ],
    )
    def k(table_hbm, idx_hbm, out_hbm, idx_v, rows_v, gsem):
        wid = lax.axis_index("s") * num_cores + lax.axis_index("c")
        base = wid * (n_chunks * _C)
        pltpu.sync_copy(idx_hbm.at[wid], idx_v)

        def body(j, carry):
            pltpu.async_copy(table_hbm.at[idx_v.at[j]], rows_v, gsem).wait()
            pltpu.sync_copy(rows_v, out_hbm.at[pl.ds(base + j * _C, _C)])
            return carry

        lax.fori_loop(0, n_chunks, body, 0)

    return k


def kernel(input, lengths, emb_table):
    b, h = input.shape
    v, d = emb_table.shape
    n = b * h
    info = plsc.get_sparse_core_info()
    n_workers = info.num_cores * info.num_subcores
    n_chunks = n // (n_workers * _C)
    idx = input.reshape(n_workers, n_chunks, _C)
    out = _gather_kernel(n, d, n_workers, n_chunks, info.num_cores)(emb_table, idx)
    return (out.reshape(b, h, d), lengths)


# speedup vs baseline: 1.4772x; 1.1317x over previous
import functools

import jax
import jax.numpy as jnp
from jax import lax
from jax.experimental import pallas as pl
from jax.experimental.pallas import tpu as pltpu
from jax.experimental.pallas import tpu_sc as plsc

_C = 1024


def _gather_kernel(n_rows, d, n_workers, n_chunks, num_cores):
    @functools.partial(
        pl.kernel,
        mesh=plsc.VectorSubcoreMesh(core_axis_name="c", subcore_axis_name="s"),
        out_type=jax.ShapeDtypeStruct((n_rows, d), jnp.float32),
        compiler_params=pltpu.CompilerParams(use_tc_tiling_on_sc=False),
        scratch_types=[
            pltpu.VMEM((n_chunks, _C), jnp.int32),
            pltpu.VMEM((_C, d), jnp.float32),
            pltpu.SemaphoreType.DMA,
        ],
    )
    def k(table_hbm, idx_hbm, out_hbm, idx_v, rows_v, gsem):
        wid = lax.axis_index("s") * num_cores + lax.axis_index("c")
        base = wid * (n_chunks * _C)
        pltpu.sync_copy(idx_hbm.at[wid], idx_v)

        def body(j, carry):
            pltpu.async_copy(table_hbm.at[idx_v.at[j]], rows_v, gsem).wait()
            pltpu.sync_copy(rows_v, out_hbm.at[pl.ds(base + j * _C, _C)])
            return carry

        lax.fori_loop(0, n_chunks, body, 0)

    return k


def kernel(input, lengths, emb_table):
    b, h = input.shape
    v, d = emb_table.shape
    n = b * h
    info = plsc.get_sparse_core_info()
    n_workers = info.num_cores * info.num_subcores
    n_chunks = n // (n_workers * _C)
    idx = input.reshape(n_workers, n_chunks, _C)
    out = _gather_kernel(n, d, n_workers, n_chunks, info.num_cores)(emb_table, idx)
    return (out.reshape(b, h, d), lengths)


# speedup vs baseline: 1.4938x; 1.0112x over previous
import functools

import jax
import jax.numpy as jnp
from jax import lax
from jax.experimental import pallas as pl
from jax.experimental.pallas import tpu as pltpu
from jax.experimental.pallas import tpu_sc as plsc

_C = 1280


def _gather_kernel(n_rows, d, n_chunks, num_cores):
    @functools.partial(
        pl.kernel,
        mesh=plsc.VectorSubcoreMesh(core_axis_name="c", subcore_axis_name="s"),
        out_type=jax.ShapeDtypeStruct((n_rows, d), jnp.float32),
        compiler_params=pltpu.CompilerParams(use_tc_tiling_on_sc=False),
        scratch_types=[
            pltpu.VMEM((n_chunks, _C), jnp.int32),
            pltpu.VMEM((_C, d), jnp.float32),
            pltpu.VMEM((_C, d), jnp.float32),
            pltpu.SemaphoreType.DMA,
            pltpu.SemaphoreType.DMA,
            pltpu.SemaphoreType.DMA,
        ],
    )
    def k(table_hbm, idx_hbm, out_hbm, idx_v, rows0, rows1, gsem, osem0, osem1):
        wid = lax.axis_index("s") * num_cores + lax.axis_index("c")
        base = wid * (n_chunks * _C)
        pltpu.sync_copy(idx_hbm.at[wid], idx_v)

        def gather(j, buf):
            pltpu.async_copy(table_hbm.at[idx_v.at[j]], buf, gsem)

        def wait_gather(buf):
            pltpu.make_async_copy(out_hbm.at[pl.ds(0, _C)], buf, gsem).wait()

        def write(j, buf, sem):
            pltpu.async_copy(buf, out_hbm.at[pl.ds(base + j * _C, _C)], sem)

        def wait_write(buf, sem):
            pltpu.make_async_copy(out_hbm.at[pl.ds(0, _C)], buf, sem).wait()

        gather(0, rows0)

        def body(i, carry):
            g0 = 2 * i
            wait_gather(rows0)

            @pl.when(i > 0)
            def _():
                wait_write(rows1, osem1)

            gather(g0 + 1, rows1)
            write(g0, rows0, osem0)
            wait_gather(rows1)
            wait_write(rows0, osem0)

            @pl.when(g0 + 2 < n_chunks)
            def _():
                gather(g0 + 2, rows0)

            write(g0 + 1, rows1, osem1)
            return carry

        lax.fori_loop(0, n_chunks // 2, body, 0)
        wait_write(rows1, osem1)

    return k


def kernel(input, lengths, emb_table):
    b, h = input.shape
    v, d = emb_table.shape
    n = b * h
    info = plsc.get_sparse_core_info()
    n_workers = info.num_cores * info.num_subcores
    n_chunks = n // (n_workers * _C)
    idx = input.reshape(n_workers, n_chunks, _C)
    out = _gather_kernel(n, d, n_chunks, info.num_cores)(emb_table, idx)
    return (out.reshape(b, h, d), lengths)
